# baseline (speedup 1.0000x reference)
import functools

import jax
import jax.numpy as jnp
from jax import lax
from jax.experimental import pallas as pl
from jax.experimental.pallas import tpu as pltpu
from jax.experimental.pallas import tpu_sc as plsc

T = 2048
D = 1024
E = 8
FF = 2048
K = 2

BT = 256
SPAD = T * K + E * BT
NB = SPAD // BT
FB2 = 512
FJ2 = FF // FB2
SFB = 512
SFJ = FF // SFB
CH = 256

NC = 2
NS = 16
NW = NC * NS
TPW = T // NW
CC = 32


def _k1_body(hr, rwr, rbr, sgr, sur, sdr,
             sh_ref, p1_ref, p2_ref, w1_ref, w2_ref, blk_ref, sel_s, rank_s):
    j = pl.program_id(0)
    h = hr[...]

    @pl.when(j == 0)
    def _router():
        scores = jax.nn.sigmoid(
            lax.dot_general(h, rwr[...], (((1,), (1,)), ((), ())),
                            preferred_element_type=jnp.float32))
        s = scores + rbr[...]
        lane = lax.broadcasted_iota(jnp.int32, (T, E), 1)
        m1 = jnp.max(s, axis=1, keepdims=True)
        i1 = jnp.min(jnp.where(s == m1, lane, E), axis=1, keepdims=True)
        mask1 = lane == i1
        s2 = jnp.where(mask1, -jnp.inf, s)
        m2 = jnp.max(s2, axis=1, keepdims=True)
        i2 = jnp.min(jnp.where(s2 == m2, lane, E), axis=1, keepdims=True)
        mask2 = lane == i2
        w1 = jnp.sum(jnp.where(mask1, scores, 0.0), axis=1, keepdims=True)
        w2 = jnp.sum(jnp.where(mask2, scores, 0.0), axis=1, keepdims=True)
        den = w1 + w2 + 1e-20
        w1_ref[...] = w1 / den
        w2_ref[...] = w2 / den

        sel_s[...] = jnp.where(mask1 | mask2, 1.0, 0.0)

        li = lax.broadcasted_iota(jnp.int32, (CH, CH), 0)
        lj = lax.broadcasted_iota(jnp.int32, (CH, CH), 1)
        ltri = jnp.where(li > lj, 1.0, 0.0)

        def chunk(c, base):
            selc = sel_s[pl.ds(c * CH, CH), :]
            rank_s[pl.ds(c * CH, CH), :] = base + lax.dot_general(
                ltri, selc, (((1,), (0,)), ((), ())),
                preferred_element_type=jnp.float32)
            return base + jnp.sum(selc, axis=0, keepdims=True)

        counts = lax.fori_loop(0, T // CH, chunk, jnp.zeros((1, E)))

        cnt_pad = jnp.floor((counts + (BT - 1)) * (1.0 / BT)) * float(BT)
        ui = lax.broadcasted_iota(jnp.int32, (E, E), 0)
        uj = lax.broadcasted_iota(jnp.int32, (E, E), 1)
        ustrict = jnp.where(ui < uj, 1.0, 0.0)
        offe = lax.dot_general(cnt_pad, ustrict, (((1,), (0,)), ((), ())),
                               preferred_element_type=jnp.float32)

        rank = rank_s[...]
        rank1 = jnp.sum(jnp.where(mask1, rank, 0.0), axis=1, keepdims=True)
        rank2 = jnp.sum(jnp.where(mask2, rank, 0.0), axis=1, keepdims=True)
        off1 = jnp.sum(jnp.where(mask1, offe, 0.0), axis=1, keepdims=True)
        off2 = jnp.sum(jnp.where(mask2, offe, 0.0), axis=1, keepdims=True)
        p1_ref[...] = (rank1 + off1).astype(jnp.int32)
        p2_ref[...] = (rank2 + off2).astype(jnp.int32)

        bstart = (lax.broadcasted_iota(jnp.int32, (NB, E), 0)
                  .astype(jnp.float32) * float(BT))
        ge = jnp.where(bstart >= offe, 1.0, 0.0)
        blk_ref[...] = (jnp.sum(ge, axis=1, keepdims=True) - 1.0
                        ).astype(jnp.int32)

    g = lax.dot_general(h, sgr[...], (((1,), (1,)), ((), ())),
                        preferred_element_type=jnp.float32)
    u = lax.dot_general(h, sur[...], (((1,), (1,)), ((), ())),
                        preferred_element_type=jnp.float32)
    a = (g * jax.nn.sigmoid(g)) * u
    p = lax.dot_general(a, sdr[...], (((1,), (1,)), ((), ())),
                        preferred_element_type=jnp.float32)

    @pl.when(j == 0)
    def _first():
        sh_ref[...] = p

    @pl.when(j > 0)
    def _rest():
        sh_ref[...] += p


def _k1(h, router_w, rb, sh_gate_w, sh_up_w, sh_down_w):
    return pl.pallas_call(
        _k1_body,
        grid=(SFJ,),
        in_specs=[
            pl.BlockSpec((T, D), lambda j: (0, 0)),
            pl.BlockSpec((E, D), lambda j: (0, 0)),
            pl.BlockSpec((1, E), lambda j: (0, 0)),
            pl.BlockSpec((SFB, D), lambda j: (j, 0)),
            pl.BlockSpec((SFB, D), lambda j: (j, 0)),
            pl.BlockSpec((D, SFB), lambda j: (0, j)),
        ],
        out_specs=[
            pl.BlockSpec((T, D), lambda j: (0, 0)),
            pl.BlockSpec((T, 1), lambda j: (0, 0)),
            pl.BlockSpec((T, 1), lambda j: (0, 0)),
            pl.BlockSpec((T, 1), lambda j: (0, 0)),
            pl.BlockSpec((T, 1), lambda j: (0, 0)),
            pl.BlockSpec((NB, 1), lambda j: (0, 0)),
        ],
        out_shape=[
            jax.ShapeDtypeStruct((T, D), jnp.float32),
            jax.ShapeDtypeStruct((T, 1), jnp.int32),
            jax.ShapeDtypeStruct((T, 1), jnp.int32),
            jax.ShapeDtypeStruct((T, 1), jnp.float32),
            jax.ShapeDtypeStruct((T, 1), jnp.float32),
            jax.ShapeDtypeStruct((NB, 1), jnp.int32),
        ],
        scratch_shapes=[pltpu.VMEM((T, E), jnp.float32),
                        pltpu.VMEM((T, E), jnp.float32)],
        compiler_params=pltpu.CompilerParams(
            dimension_semantics=("arbitrary",)),
    )(h, router_w, rb, sh_gate_w, sh_up_w, sh_down_w)


_MESH = plsc.VectorSubcoreMesh(core_axis_name="c", subcore_axis_name="s")


@functools.partial(
    pl.kernel,
    out_type=(jax.ShapeDtypeStruct((SPAD, D), jnp.float32),
              jax.ShapeDtypeStruct((SPAD,), jnp.float32)),
    mesh=_MESH,
    scratch_types=[
        pltpu.VMEM((TPW, D), jnp.float32),
        pltpu.VMEM((TPW,), jnp.int32),
        pltpu.VMEM((TPW,), jnp.int32),
        pltpu.VMEM((TPW,), jnp.float32),
        pltpu.VMEM((TPW,), jnp.float32),
    ],
)
def _sc_dispatch(p1_hbm, p2_hbm, w1_hbm, w2_hbm, h_hbm, hs_hbm, ws_hbm,
                 hv, p1v, p2v, w1v, w2v):
    wid = lax.axis_index("s") * NC + lax.axis_index("c")
    baset = wid * TPW
    pltpu.sync_copy(h_hbm.at[pl.ds(baset, TPW)], hv)
    pltpu.sync_copy(p1_hbm.at[pl.ds(baset, TPW)], p1v)
    pltpu.sync_copy(p2_hbm.at[pl.ds(baset, TPW)], p2v)
    pltpu.sync_copy(w1_hbm.at[pl.ds(baset, TPW)], w1v)
    pltpu.sync_copy(w2_hbm.at[pl.ds(baset, TPW)], w2v)
    pltpu.sync_copy(hv, hs_hbm.at[p1v])
    pltpu.sync_copy(hv, hs_hbm.at[p2v])
    pltpu.sync_copy(w1v, ws_hbm.at[p1v])
    pltpu.sync_copy(w2v, ws_hbm.at[p2v])


def _k2_body(blk_ref, hsr, wsr, gwr, uwr, dwr, ys_ref, acc_ref):
    j = pl.program_id(0)
    b = pl.program_id(1)
    hsb = hsr[...]
    g = lax.dot_general(hsb, gwr[0], (((1,), (1,)), ((), ())),
                        preferred_element_type=jnp.float32)
    u = lax.dot_general(hsb, uwr[0], (((1,), (1,)), ((), ())),
                        preferred_element_type=jnp.float32)
    a = ((g * jax.nn.sigmoid(g)) * u) * wsr[0]
    p = lax.dot_general(a, dwr[0], (((1,), (1,)), ((), ())),
                        preferred_element_type=jnp.float32)

    @pl.when(j == 0)
    def _first():
        acc_ref[pl.ds(b * BT, BT), :] = p

    @pl.when(j > 0)
    def _rest():
        acc_ref[pl.ds(b * BT, BT), :] += p

    @pl.when(j == FJ2 - 1)
    def _emit():
        ys_ref[...] = acc_ref[pl.ds(b * BT, BT), :]


def _k2(blk, hs, ws3, gate_w, up_w, down_w):
    return pl.pallas_call(
        _k2_body,
        grid_spec=pltpu.PrefetchScalarGridSpec(
            num_scalar_prefetch=1,
            grid=(FJ2, NB),
            in_specs=[
                pl.BlockSpec((BT, D), lambda j, b, blk: (b, 0)),
                pl.BlockSpec((1, BT, 1), lambda j, b, blk: (b, 0, 0)),
                pl.BlockSpec((1, FB2, D), lambda j, b, blk: (blk[b], j, 0)),
                pl.BlockSpec((1, FB2, D), lambda j, b, blk: (blk[b], j, 0)),
                pl.BlockSpec((1, D, FB2), lambda j, b, blk: (blk[b], 0, j)),
            ],
            out_specs=pl.BlockSpec(
                (BT, D), lambda j, b, blk: (jnp.where(j == FJ2 - 1, b, 0), 0)),
            scratch_shapes=[pltpu.VMEM((SPAD, D), jnp.float32)],
        ),
        out_shape=jax.ShapeDtypeStruct((SPAD, D), jnp.float32),
        compiler_params=pltpu.CompilerParams(
            dimension_semantics=("arbitrary", "arbitrary")),
    )(blk, hs, ws3, gate_w, up_w, down_w)


@functools.partial(
    pl.kernel,
    out_type=jax.ShapeDtypeStruct((T, D), jnp.float32),
    mesh=_MESH,
    scratch_types=[
        pltpu.VMEM((CC,), jnp.int32),
        pltpu.VMEM((CC,), jnp.int32),
        pltpu.VMEM((CC, D), jnp.float32),
        pltpu.VMEM((CC, D), jnp.float32),
        pltpu.VMEM((CC, D), jnp.float32),
    ],
)
def _sc_combine(ys_hbm, sh_hbm, p1_hbm, p2_hbm, out_hbm,
                i1v, i2v, b1, b2, b3):
    wid = lax.axis_index("s") * NC + lax.axis_index("c")
    baset = wid * TPW
    for c in range(TPW // CC):
        pltpu.sync_copy(p1_hbm.at[pl.ds(baset + c * CC, CC)], i1v)
        pltpu.sync_copy(p2_hbm.at[pl.ds(baset + c * CC, CC)], i2v)
        pltpu.sync_copy(ys_hbm.at[i1v], b1)
        pltpu.sync_copy(ys_hbm.at[i2v], b2)
        pltpu.sync_copy(sh_hbm.at[pl.ds(baset + c * CC, CC)], b3)

        def addrow(rr, carry):
            for cc in range(D // 16):
                sl = pl.ds(cc * 16, 16)
                b3[rr, sl] = b3[rr, sl] + b1[rr, sl] + b2[rr, sl]
            return carry

        lax.fori_loop(0, CC, addrow, 0)
        pltpu.sync_copy(b3, out_hbm.at[pl.ds(baset + c * CC, CC)])


def kernel(x, router_w, router_bias, gate_w, up_w, down_w,
           sh_gate_w, sh_up_w, sh_down_w):
    h = x.reshape(T, D)
    rb = router_bias.reshape(1, E)
    sh, p1, p2, w1, w2, blk = _k1(h, router_w, rb,
                                  sh_gate_w, sh_up_w, sh_down_w)
    p1f = p1.reshape(T)
    p2f = p2.reshape(T)
    hs, ws = _sc_dispatch(p1f, p2f, w1.reshape(T), w2.reshape(T), h)
    ys = _k2(blk.reshape(NB), hs, ws.reshape(NB, BT, 1),
             gate_w, up_w, down_w)
    out = _sc_combine(ys, sh, p1f, p2f)
    return out.reshape(x.shape)


# speedup vs baseline: 1.2795x; 1.2795x over previous
import functools

import jax
import jax.numpy as jnp
from jax import lax
from jax.experimental import pallas as pl
from jax.experimental.pallas import tpu as pltpu
from jax.experimental.pallas import tpu_sc as plsc

T = 2048
D = 1024
E = 8
FF = 2048
K = 2

BT = 256
SPAD = T * K + E * BT
NB = SPAD // BT
FB2 = 1024
FJ2 = FF // FB2
SFB = 512
SFJ = FF // SFB
CH = 256

NC = 2
NS = 16
NW = NC * NS
TPW = T // NW
CC = 32


def _k1_body(hr, rwr, rbr,
             p1_ref, p2_ref, w1_ref, w2_ref, blk_ref, sel_s, rank_s):
    h = hr[...]

    if True:
        scores = jax.nn.sigmoid(
            lax.dot_general(h, rwr[...], (((1,), (1,)), ((), ())),
                            preferred_element_type=jnp.float32))
        s = scores + rbr[...]
        lane = lax.broadcasted_iota(jnp.int32, (T, E), 1)
        m1 = jnp.max(s, axis=1, keepdims=True)
        i1 = jnp.min(jnp.where(s == m1, lane, E), axis=1, keepdims=True)
        mask1 = lane == i1
        s2 = jnp.where(mask1, -jnp.inf, s)
        m2 = jnp.max(s2, axis=1, keepdims=True)
        i2 = jnp.min(jnp.where(s2 == m2, lane, E), axis=1, keepdims=True)
        mask2 = lane == i2
        w1 = jnp.sum(jnp.where(mask1, scores, 0.0), axis=1, keepdims=True)
        w2 = jnp.sum(jnp.where(mask2, scores, 0.0), axis=1, keepdims=True)
        den = w1 + w2 + 1e-20
        w1_ref[...] = w1 / den
        w2_ref[...] = w2 / den

        sel_s[...] = jnp.where(mask1 | mask2, 1.0, 0.0)

        li = lax.broadcasted_iota(jnp.int32, (CH, CH), 0)
        lj = lax.broadcasted_iota(jnp.int32, (CH, CH), 1)
        ltri = jnp.where(li > lj, 1.0, 0.0)

        def chunk(c, base):
            selc = sel_s[pl.ds(c * CH, CH), :]
            rank_s[pl.ds(c * CH, CH), :] = base + lax.dot_general(
                ltri, selc, (((1,), (0,)), ((), ())),
                preferred_element_type=jnp.float32)
            return base + jnp.sum(selc, axis=0, keepdims=True)

        counts = lax.fori_loop(0, T // CH, chunk, jnp.zeros((1, E)))

        cnt_pad = jnp.floor((counts + (BT - 1)) * (1.0 / BT)) * float(BT)
        ui = lax.broadcasted_iota(jnp.int32, (E, E), 0)
        uj = lax.broadcasted_iota(jnp.int32, (E, E), 1)
        ustrict = jnp.where(ui < uj, 1.0, 0.0)
        offe = lax.dot_general(cnt_pad, ustrict, (((1,), (0,)), ((), ())),
                               preferred_element_type=jnp.float32)

        rank = rank_s[...]
        rank1 = jnp.sum(jnp.where(mask1, rank, 0.0), axis=1, keepdims=True)
        rank2 = jnp.sum(jnp.where(mask2, rank, 0.0), axis=1, keepdims=True)
        off1 = jnp.sum(jnp.where(mask1, offe, 0.0), axis=1, keepdims=True)
        off2 = jnp.sum(jnp.where(mask2, offe, 0.0), axis=1, keepdims=True)
        p1_ref[...] = (rank1 + off1).astype(jnp.int32)
        p2_ref[...] = (rank2 + off2).astype(jnp.int32)

        bstart = (lax.broadcasted_iota(jnp.int32, (NB, E), 0)
                  .astype(jnp.float32) * float(BT))
        ge = jnp.where(bstart >= offe, 1.0, 0.0)
        blk_ref[...] = (jnp.sum(ge, axis=1, keepdims=True) - 1.0
                        ).astype(jnp.int32)


def _k1(h, router_w, rb):
    return pl.pallas_call(
        _k1_body,
        grid=(1,),
        in_specs=[
            pl.BlockSpec((T, D), lambda j: (0, 0)),
            pl.BlockSpec((E, D), lambda j: (0, 0)),
            pl.BlockSpec((1, E), lambda j: (0, 0)),
        ],
        out_specs=[
            pl.BlockSpec((T, 1), lambda j: (0, 0)),
            pl.BlockSpec((T, 1), lambda j: (0, 0)),
            pl.BlockSpec((T, 1), lambda j: (0, 0)),
            pl.BlockSpec((T, 1), lambda j: (0, 0)),
            pl.BlockSpec((NB, 1), lambda j: (0, 0)),
        ],
        out_shape=[
            jax.ShapeDtypeStruct((T, 1), jnp.int32),
            jax.ShapeDtypeStruct((T, 1), jnp.int32),
            jax.ShapeDtypeStruct((T, 1), jnp.float32),
            jax.ShapeDtypeStruct((T, 1), jnp.float32),
            jax.ShapeDtypeStruct((NB, 1), jnp.int32),
        ],
        scratch_shapes=[pltpu.VMEM((T, E), jnp.float32),
                        pltpu.VMEM((T, E), jnp.float32)],
        compiler_params=pltpu.CompilerParams(
            dimension_semantics=("arbitrary",)),
    )(h, router_w, rb)


def _ksh_body(hr, sgr, sur, sdr, sh_ref):
    j = pl.program_id(0)
    h = hr[...]
    g = lax.dot_general(h, sgr[...], (((1,), (1,)), ((), ())),
                        preferred_element_type=jnp.float32)
    u = lax.dot_general(h, sur[...], (((1,), (1,)), ((), ())),
                        preferred_element_type=jnp.float32)
    a = (g * jax.nn.sigmoid(g)) * u
    p = lax.dot_general(a, sdr[...], (((1,), (1,)), ((), ())),
                        preferred_element_type=jnp.float32)

    @pl.when(j == 0)
    def _first():
        sh_ref[...] = p

    @pl.when(j > 0)
    def _rest():
        sh_ref[...] += p


def _ksh(h, sh_gate_w, sh_up_w, sh_down_w):
    return pl.pallas_call(
        _ksh_body,
        grid=(SFJ,),
        in_specs=[
            pl.BlockSpec((T, D), lambda j: (0, 0)),
            pl.BlockSpec((SFB, D), lambda j: (j, 0)),
            pl.BlockSpec((SFB, D), lambda j: (j, 0)),
            pl.BlockSpec((D, SFB), lambda j: (0, j)),
        ],
        out_specs=pl.BlockSpec((T, D), lambda j: (0, 0)),
        out_shape=jax.ShapeDtypeStruct((T, D), jnp.float32),
        compiler_params=pltpu.CompilerParams(
            dimension_semantics=("arbitrary",)),
    )(h, sh_gate_w, sh_up_w, sh_down_w)


_MESH = plsc.VectorSubcoreMesh(core_axis_name="c", subcore_axis_name="s")


@functools.partial(
    pl.kernel,
    out_type=(jax.ShapeDtypeStruct((SPAD, D), jnp.float32),
              jax.ShapeDtypeStruct((SPAD,), jnp.float32)),
    mesh=_MESH,
    scratch_types=[
        pltpu.VMEM((TPW, D), jnp.float32),
        pltpu.VMEM((TPW,), jnp.int32),
        pltpu.VMEM((TPW,), jnp.int32),
        pltpu.VMEM((TPW,), jnp.float32),
        pltpu.VMEM((TPW,), jnp.float32),
    ],
)
def _sc_dispatch(p1_hbm, p2_hbm, w1_hbm, w2_hbm, h_hbm, hs_hbm, ws_hbm,
                 hv, p1v, p2v, w1v, w2v):
    wid = lax.axis_index("s") * NC + lax.axis_index("c")
    baset = wid * TPW
    pltpu.sync_copy(h_hbm.at[pl.ds(baset, TPW)], hv)
    pltpu.sync_copy(p1_hbm.at[pl.ds(baset, TPW)], p1v)
    pltpu.sync_copy(p2_hbm.at[pl.ds(baset, TPW)], p2v)
    pltpu.sync_copy(w1_hbm.at[pl.ds(baset, TPW)], w1v)
    pltpu.sync_copy(w2_hbm.at[pl.ds(baset, TPW)], w2v)
    pltpu.sync_copy(hv, hs_hbm.at[p1v])
    pltpu.sync_copy(hv, hs_hbm.at[p2v])
    pltpu.sync_copy(w1v, ws_hbm.at[p1v])
    pltpu.sync_copy(w2v, ws_hbm.at[p2v])


def _k2_body(blk_ref, hsr, wsr, gwr, uwr, dwr, ys_ref, acc_ref):
    j = pl.program_id(0)
    b = pl.program_id(1)
    hsb = hsr[...]
    g = lax.dot_general(hsb, gwr[0], (((1,), (1,)), ((), ())),
                        preferred_element_type=jnp.float32)
    u = lax.dot_general(hsb, uwr[0], (((1,), (1,)), ((), ())),
                        preferred_element_type=jnp.float32)
    a = ((g * jax.nn.sigmoid(g)) * u) * wsr[0]
    p = lax.dot_general(a, dwr[0], (((1,), (1,)), ((), ())),
                        preferred_element_type=jnp.float32)

    @pl.when(j == 0)
    def _first():
        acc_ref[pl.ds(b * BT, BT), :] = p

    @pl.when(j > 0)
    def _rest():
        acc_ref[pl.ds(b * BT, BT), :] += p

    @pl.when(j == FJ2 - 1)
    def _emit():
        ys_ref[...] = acc_ref[pl.ds(b * BT, BT), :]


def _k2(blk, hs, ws3, gate_w, up_w, down_w):
    return pl.pallas_call(
        _k2_body,
        grid_spec=pltpu.PrefetchScalarGridSpec(
            num_scalar_prefetch=1,
            grid=(FJ2, NB),
            in_specs=[
                pl.BlockSpec((BT, D), lambda j, b, blk: (b, 0)),
                pl.BlockSpec((1, BT, 1), lambda j, b, blk: (b, 0, 0)),
                pl.BlockSpec((1, FB2, D), lambda j, b, blk: (blk[b], j, 0)),
                pl.BlockSpec((1, FB2, D), lambda j, b, blk: (blk[b], j, 0)),
                pl.BlockSpec((1, D, FB2), lambda j, b, blk: (blk[b], 0, j)),
            ],
            out_specs=pl.BlockSpec(
                (BT, D), lambda j, b, blk: (jnp.where(j == FJ2 - 1, b, 0), 0)),
            scratch_shapes=[pltpu.VMEM((SPAD, D), jnp.float32)],
        ),
        out_shape=jax.ShapeDtypeStruct((SPAD, D), jnp.float32),
        compiler_params=pltpu.CompilerParams(
            dimension_semantics=("arbitrary", "arbitrary")),
    )(blk, hs, ws3, gate_w, up_w, down_w)


@functools.partial(
    pl.kernel,
    out_type=jax.ShapeDtypeStruct((T, D), jnp.float32),
    mesh=_MESH,
    scratch_types=[
        pltpu.VMEM((CC,), jnp.int32),
        pltpu.VMEM((CC,), jnp.int32),
        pltpu.VMEM((CC, D), jnp.float32),
        pltpu.VMEM((CC, D), jnp.float32),
        pltpu.VMEM((CC, D), jnp.float32),
    ],
)
def _sc_combine(ys_hbm, sh_hbm, p1_hbm, p2_hbm, out_hbm,
                i1v, i2v, b1, b2, b3):
    wid = lax.axis_index("s") * NC + lax.axis_index("c")
    baset = wid * TPW
    for c in range(TPW // CC):
        pltpu.sync_copy(p1_hbm.at[pl.ds(baset + c * CC, CC)], i1v)
        pltpu.sync_copy(p2_hbm.at[pl.ds(baset + c * CC, CC)], i2v)
        pltpu.sync_copy(ys_hbm.at[i1v], b1)
        pltpu.sync_copy(ys_hbm.at[i2v], b2)
        pltpu.sync_copy(sh_hbm.at[pl.ds(baset + c * CC, CC)], b3)

        def addrow(rr, carry):
            for cc in range(D // 16):
                sl = pl.ds(cc * 16, 16)
                b3[rr, sl] = b3[rr, sl] + b1[rr, sl] + b2[rr, sl]
            return carry

        lax.fori_loop(0, CC, addrow, 0)
        pltpu.sync_copy(b3, out_hbm.at[pl.ds(baset + c * CC, CC)])


def kernel(x, router_w, router_bias, gate_w, up_w, down_w,
           sh_gate_w, sh_up_w, sh_down_w):
    h = x.reshape(T, D)
    rb = router_bias.reshape(1, E)
    p1, p2, w1, w2, blk = _k1(h, router_w, rb)
    p1f = p1.reshape(T)
    p2f = p2.reshape(T)
    hs, ws = _sc_dispatch(p1f, p2f, w1.reshape(T), w2.reshape(T), h)
    sh = _ksh(h, sh_gate_w, sh_up_w, sh_down_w)
    ys = _k2(blk.reshape(NB), hs, ws.reshape(NB, BT, 1),
             gate_w, up_w, down_w)
    out = _sc_combine(ys, sh, p1f, p2f)
    return out.reshape(x.shape)
